# baseline (speedup 1.0000x reference)
import functools

import jax
import jax.numpy as jnp
from jax.experimental import pallas as pl
from jax.experimental.pallas import tpu as pltpu

N_ROWS = 10000
D = 128
CHUNK = 1000
NC = N_ROWS // CHUNK
INV_N = 1.0 / N_ROWS


def _body(
    xs_hbm,
    xt_hbm,
    w_ref,
    hs_hbm,
    ht_hbm,
    loss_ref,
    xs_buf,
    xt_buf,
    hs_buf,
    ht_buf,
    acc_ref,
    sem_xs,
    sem_xt,
    sem_hs,
    sem_ht,
):
    def in_copy(hbm, buf, sem, k):
        slot = jax.lax.rem(k, 2)
        return pltpu.make_async_copy(
            hbm.at[pl.ds(k * CHUNK, CHUNK), :], buf.at[slot], sem.at[slot]
        )

    def out_copy(hbm, buf, sem, k):
        slot = jax.lax.rem(k, 2)
        return pltpu.make_async_copy(
            buf.at[slot], hbm.at[pl.ds(k * CHUNK, CHUNK), :], sem.at[slot]
        )

    in_copy(xs_hbm, xs_buf, sem_xs, 0).start()
    in_copy(xt_hbm, xt_buf, sem_xt, 0).start()
    in_copy(xs_hbm, xs_buf, sem_xs, 1).start()
    in_copy(xt_hbm, xt_buf, sem_xt, 1).start()
    acc_ref[...] = jnp.zeros_like(acc_ref)

    def step(k, carry):
        slot = k % 2
        in_copy(xs_hbm, xs_buf, sem_xs, k).wait()
        in_copy(xt_hbm, xt_buf, sem_xt, k).wait()

        if k >= 2:
            out_copy(hs_hbm, hs_buf, sem_hs, k - 2).wait()
            out_copy(ht_hbm, ht_buf, sem_ht, k - 2).wait()

        def run(x_buf, w, h_buf, base):
            h = jnp.dot(
                x_buf[slot], w, preferred_element_type=jnp.float32
            )
            h_buf[slot] = h
            h2 = h * h
            h3 = h2 * h
            h4 = h2 * h2
            h5 = h4 * h
            part = jnp.concatenate(
                [
                    jnp.sum(h, axis=0, keepdims=True),
                    jnp.sum(h2, axis=0, keepdims=True),
                    jnp.sum(h3, axis=0, keepdims=True),
                    jnp.sum(h4, axis=0, keepdims=True),
                    jnp.sum(h5, axis=0, keepdims=True),
                ],
                axis=0,
            )
            acc_ref[base : base + 5, :] += part

        run(xs_buf, w_ref[0], hs_buf, 0)
        run(xt_buf, w_ref[1], ht_buf, 8)

        out_copy(hs_hbm, hs_buf, sem_hs, k).start()
        out_copy(ht_hbm, ht_buf, sem_ht, k).start()

        if k + 2 < NC:
            in_copy(xs_hbm, xs_buf, sem_xs, k + 2).start()
            in_copy(xt_hbm, xt_buf, sem_xt, k + 2).start()

        return carry

    for k in range(NC):
        step(k, 0)

    a = acc_ref[...] * INV_N

    def central(rows):
        m1 = rows[0:1, :]
        m2 = rows[1:2, :]
        m3 = rows[2:3, :]
        m4 = rows[3:4, :]
        m5 = rows[4:5, :]
        c2 = m2 - m1 * m1
        c3 = m3 - 3.0 * m1 * m2 + 2.0 * m1**3
        c4 = m4 - 4.0 * m1 * m3 + 6.0 * m1**2 * m2 - 3.0 * m1**4
        c5 = (
            m5
            - 5.0 * m1 * m4
            + 10.0 * m1**2 * m3
            - 10.0 * m1**3 * m2
            + 4.0 * m1**5
        )
        return m1, c2, c3, c4, c5

    s_moms = central(a[0:5, :])
    t_moms = central(a[8:13, :])
    loss = jnp.zeros((1, 1), jnp.float32)
    for s_m, t_m in zip(s_moms, t_moms):
        d = s_m - t_m
        loss = loss + jnp.sqrt(jnp.sum(d * d, keepdims=True))
    loss_ref[...] = loss

    out_copy(hs_hbm, hs_buf, sem_hs, NC - 2).wait()
    out_copy(ht_hbm, ht_buf, sem_ht, NC - 2).wait()
    out_copy(hs_hbm, hs_buf, sem_hs, NC - 1).wait()
    out_copy(ht_hbm, ht_buf, sem_ht, NC - 1).wait()


@functools.partial(jax.jit, static_argnames=())
def _run(feat_s, feat_t, w_stacked):
    kernel_fn = pl.pallas_call(
        _body,
        in_specs=[
            pl.BlockSpec(memory_space=pltpu.HBM),
            pl.BlockSpec(memory_space=pltpu.HBM),
            pl.BlockSpec(memory_space=pltpu.VMEM),
        ],
        out_specs=[
            pl.BlockSpec(memory_space=pltpu.HBM),
            pl.BlockSpec(memory_space=pltpu.HBM),
            pl.BlockSpec(memory_space=pltpu.VMEM),
        ],
        out_shape=[
            jax.ShapeDtypeStruct((N_ROWS, D), jnp.float32),
            jax.ShapeDtypeStruct((N_ROWS, D), jnp.float32),
            jax.ShapeDtypeStruct((1, 1), jnp.float32),
        ],
        scratch_shapes=[
            pltpu.VMEM((2, CHUNK, D), jnp.float32),
            pltpu.VMEM((2, CHUNK, D), jnp.float32),
            pltpu.VMEM((2, CHUNK, D), jnp.float32),
            pltpu.VMEM((2, CHUNK, D), jnp.float32),
            pltpu.VMEM((16, D), jnp.float32),
            pltpu.SemaphoreType.DMA((2,)),
            pltpu.SemaphoreType.DMA((2,)),
            pltpu.SemaphoreType.DMA((2,)),
            pltpu.SemaphoreType.DMA((2,)),
        ],
    )
    return kernel_fn(feat_s, feat_t, w_stacked)


def kernel(feat_s, feat_t, W_s, W_t, edge_index):
    del edge_index
    w_stacked = jnp.stack([W_s, W_t])
    h_s, h_t, loss = _run(feat_s, feat_t, w_stacked)
    return (h_s, h_t, loss[0, 0])


# speedup vs baseline: 1.3724x; 1.3724x over previous
import functools

import jax
import jax.numpy as jnp
from jax.experimental import pallas as pl
from jax.experimental.pallas import tpu as pltpu

N_ROWS = 10000
D = 128
TILE = 5000
NJ = N_ROWS // TILE
SUB = 1000
NS = TILE // SUB
INV_N = 1.0 / N_ROWS


def _body(*refs):
    xs_refs = refs[0:NS]
    xt_refs = refs[NS : 2 * NS]
    w_ref = refs[2 * NS]
    hs_ref, ht_ref, loss_ref, acc_ref = refs[2 * NS + 1 :]

    j = pl.program_id(0)

    @pl.when(j == 0)
    def _init():
        acc_ref[...] = jnp.zeros_like(acc_ref)

    def run(x_refs, w, h_out_ref, base):
        p1 = p2 = p3 = p4 = p5 = None
        for m, x_ref in enumerate(x_refs):
            h = jnp.dot(x_ref[...], w, preferred_element_type=jnp.float32)
            h_out_ref[m * SUB : (m + 1) * SUB, :] = h
            h2 = h * h
            h3 = h2 * h
            h4 = h2 * h2
            h5 = h4 * h
            s1 = jnp.sum(h, axis=0, keepdims=True)
            s2 = jnp.sum(h2, axis=0, keepdims=True)
            s3 = jnp.sum(h3, axis=0, keepdims=True)
            s4 = jnp.sum(h4, axis=0, keepdims=True)
            s5 = jnp.sum(h5, axis=0, keepdims=True)
            if m == 0:
                p1, p2, p3, p4, p5 = s1, s2, s3, s4, s5
            else:
                p1, p2, p3, p4, p5 = p1 + s1, p2 + s2, p3 + s3, p4 + s4, p5 + s5
        part = jnp.concatenate([p1, p2, p3, p4, p5], axis=0)
        acc_ref[base : base + 5, :] += part

    run(xs_refs, w_ref[0], hs_ref, 0)
    run(xt_refs, w_ref[1], ht_ref, 8)

    @pl.when(j == NJ - 1)
    def _finish():
        a = acc_ref[...] * INV_N

        def central(rows):
            m1 = rows[0:1, :]
            m2 = rows[1:2, :]
            m3 = rows[2:3, :]
            m4 = rows[3:4, :]
            m5 = rows[4:5, :]
            c2 = m2 - m1 * m1
            c3 = m3 - 3.0 * m1 * m2 + 2.0 * m1**3
            c4 = m4 - 4.0 * m1 * m3 + 6.0 * m1**2 * m2 - 3.0 * m1**4
            c5 = (
                m5
                - 5.0 * m1 * m4
                + 10.0 * m1**2 * m3
                - 10.0 * m1**3 * m2
                + 4.0 * m1**5
            )
            return m1, c2, c3, c4, c5

        s_moms = central(a[0:5, :])
        t_moms = central(a[8:13, :])
        loss = jnp.zeros((1, 1), jnp.float32)
        for s_m, t_m in zip(s_moms, t_moms):
            d = s_m - t_m
            loss = loss + jnp.sqrt(jnp.sum(d * d, keepdims=True))
        loss_ref[...] = loss


def _sub_spec(m):
    return pl.BlockSpec((SUB, D), lambda j, m=m: (NS * j + m, 0))


@functools.partial(jax.jit, static_argnames=())
def _run(feat_s, feat_t, w_stacked):
    kernel_fn = pl.pallas_call(
        _body,
        grid=(NJ,),
        in_specs=(
            [_sub_spec(m) for m in range(NS)]
            + [_sub_spec(m) for m in range(NS)]
            + [pl.BlockSpec((2, D, D), lambda j: (0, 0, 0))]
        ),
        out_specs=[
            pl.BlockSpec((TILE, D), lambda j: (j, 0)),
            pl.BlockSpec((TILE, D), lambda j: (j, 0)),
            pl.BlockSpec((1, 1), lambda j: (0, 0)),
        ],
        out_shape=[
            jax.ShapeDtypeStruct((N_ROWS, D), jnp.float32),
            jax.ShapeDtypeStruct((N_ROWS, D), jnp.float32),
            jax.ShapeDtypeStruct((1, 1), jnp.float32),
        ],
        scratch_shapes=[pltpu.VMEM((16, D), jnp.float32)],
        compiler_params=pltpu.CompilerParams(
            dimension_semantics=("arbitrary",),
        ),
    )
    args = [feat_s] * NS + [feat_t] * NS + [w_stacked]
    return kernel_fn(*args)


def kernel(feat_s, feat_t, W_s, W_t, edge_index):
    del edge_index
    w_stacked = jnp.stack([W_s, W_t])
    h_s, h_t, loss = _run(feat_s, feat_t, w_stacked)
    return (h_s, h_t, loss[0, 0])


# speedup vs baseline: 1.3898x; 1.0126x over previous
import functools

import jax
import jax.numpy as jnp
from jax.experimental import pallas as pl
from jax.experimental.pallas import tpu as pltpu

N_ROWS = 10000
D = 128
TILE = 5000
NJ = N_ROWS // TILE
INV_N = 1.0 / N_ROWS


def _body(xs_ref, xt_ref, w_ref, hs_ref, ht_ref, loss_ref, acc_ref):
    j = pl.program_id(0)

    @pl.when(j == 0)
    def _init():
        acc_ref[...] = jnp.zeros_like(acc_ref)

    def run(x_ref, w, h_out_ref, base):
        h = jnp.dot(x_ref[...], w, preferred_element_type=jnp.float32)
        h_out_ref[...] = h
        h2 = h * h
        h3 = h2 * h
        h4 = h2 * h2
        h5 = h4 * h
        part = jnp.concatenate(
            [
                jnp.sum(h, axis=0, keepdims=True),
                jnp.sum(h2, axis=0, keepdims=True),
                jnp.sum(h3, axis=0, keepdims=True),
                jnp.sum(h4, axis=0, keepdims=True),
                jnp.sum(h5, axis=0, keepdims=True),
            ],
            axis=0,
        )
        acc_ref[base : base + 5, :] += part

    run(xs_ref, w_ref[0], hs_ref, 0)
    run(xt_ref, w_ref[1], ht_ref, 8)

    @pl.when(j == NJ - 1)
    def _finish():
        a = acc_ref[...] * INV_N

        def central(rows):
            m1 = rows[0:1, :]
            m2 = rows[1:2, :]
            m3 = rows[2:3, :]
            m4 = rows[3:4, :]
            m5 = rows[4:5, :]
            c2 = m2 - m1 * m1
            c3 = m3 - 3.0 * m1 * m2 + 2.0 * m1**3
            c4 = m4 - 4.0 * m1 * m3 + 6.0 * m1**2 * m2 - 3.0 * m1**4
            c5 = (
                m5
                - 5.0 * m1 * m4
                + 10.0 * m1**2 * m3
                - 10.0 * m1**3 * m2
                + 4.0 * m1**5
            )
            return m1, c2, c3, c4, c5

        s_moms = central(a[0:5, :])
        t_moms = central(a[8:13, :])
        loss = jnp.zeros((1, 1), jnp.float32)
        for s_m, t_m in zip(s_moms, t_moms):
            d = s_m - t_m
            loss = loss + jnp.sqrt(jnp.sum(d * d, keepdims=True))
        loss_ref[...] = loss


@functools.partial(jax.jit, static_argnames=())
def _run(feat_s, feat_t, w_stacked):
    kernel_fn = pl.pallas_call(
        _body,
        grid=(NJ,),
        in_specs=[
            pl.BlockSpec((TILE, D), lambda j: (j, 0)),
            pl.BlockSpec((TILE, D), lambda j: (j, 0)),
            pl.BlockSpec((2, D, D), lambda j: (0, 0, 0)),
        ],
        out_specs=[
            pl.BlockSpec((TILE, D), lambda j: (j, 0)),
            pl.BlockSpec((TILE, D), lambda j: (j, 0)),
            pl.BlockSpec((1, 1), lambda j: (0, 0)),
        ],
        out_shape=[
            jax.ShapeDtypeStruct((N_ROWS, D), jnp.float32),
            jax.ShapeDtypeStruct((N_ROWS, D), jnp.float32),
            jax.ShapeDtypeStruct((1, 1), jnp.float32),
        ],
        scratch_shapes=[pltpu.VMEM((16, D), jnp.float32)],
        compiler_params=pltpu.CompilerParams(
            dimension_semantics=("arbitrary",),
        ),
    )
    return kernel_fn(feat_s, feat_t, w_stacked)


def kernel(feat_s, feat_t, W_s, W_t, edge_index):
    del edge_index
    w_stacked = jnp.stack([W_s, W_t])
    h_s, h_t, loss = _run(feat_s, feat_t, w_stacked)
    return (h_s, h_t, loss[0, 0])
